# baseline (speedup 1.0000x reference)
import math

import jax
import jax.numpy as jnp
import numpy as np
from jax.experimental import pallas as pl
from jax.experimental.pallas import tpu as pltpu

IMG = 512
STEPS = (4, 8, 16, 32)
SCALES = (0.04, 0.1, 0.26, 0.45, 0.58)
FHW = (128, 64, 32, 16)
CH = (96, 192, 384, 768)
NPIX = tuple(f * f for f in FHW)
PBLK = 4096
NBLK_ALL = 6
NPIX_ALL = 21760
NB = 4
NCONF = NB * 2
NLOC = NB * 4
NCH = NCONF + NLOC


def _anchor_const_table() -> np.ndarray:
    tab = np.zeros((4, NB * 14), dtype=np.float32)
    for i in range(4):
        s, sn = SCALES[i], SCALES[i + 1]
        wh = [
            (s, s),
            (math.sqrt(s * sn), math.sqrt(s * sn)),
            (s * math.sqrt(2.0), s / math.sqrt(2.0)),
            (s * math.sqrt(0.5), s / math.sqrt(0.5)),
        ]
        for k in range(NB):
            base = 14 * k
            tab[i, base + 8] = wh[k][0]
            tab[i, base + 9] = wh[k][1]
            tab[i, base + 10:base + 14] = (0.1, 0.1, 0.2, 0.2)
    return tab


_CONST56 = _anchor_const_table()


def _conv_rows(feat_ref, wa, wb, wc, btab, i, r0, r1):
    fw = FHW[i]
    fh = fw
    npx = (r1 - r0) * fw
    ulo, uhi = max(0, r0 - 1), min(fh, r1 + 1)
    nrow = uhi - ulo
    nq = nrow * fw

    xs3 = feat_ref[0, ulo:uhi]
    zcol = jnp.zeros((nrow, 1, CH[i]), jnp.float32)
    xm1 = jnp.concatenate([zcol, xs3[:, 0:fw - 1, :]], axis=1)
    xp1 = jnp.concatenate([xs3[:, 1:fw, :], zcol], axis=1)

    def f(a3):
        return a3.reshape(nq, CH[i]).astype(jnp.bfloat16)

    u = (jnp.dot(f(xm1), wa, preferred_element_type=jnp.float32)
         + jnp.dot(f(xs3), wb, preferred_element_type=jnp.float32)
         + jnp.dot(f(xp1), wc, preferred_element_type=jnp.float32))

    off = r0 * fw - ulo * fw
    acc = (u[off:off + npx, NCH:2 * NCH]
           + jnp.broadcast_to(btab[i:i + 1, 0:NCH], (npx, NCH)))
    if r0 == 0:
        acc = acc + jnp.concatenate(
            [jnp.zeros((fw, NCH), jnp.float32), u[0:npx - fw, 0:NCH]], axis=0)
    else:
        acc = acc + u[off - fw:off - fw + npx, 0:NCH]
    if r1 == fh:
        acc = acc + jnp.concatenate(
            [u[off + fw:off + npx, 2 * NCH:3 * NCH],
             jnp.zeros((fw, NCH), jnp.float32)], axis=0)
    else:
        acc = acc + u[off + fw:off + fw + npx, 2 * NCH:3 * NCH]
    return acc


def _assemble(acc, ctab, i, p_base):
    n = acc.shape[0]
    fw = FHW[i]
    conf = acc[:, 0:NCONF]
    locv = acc[:, NCONF:NCH]
    lane = jax.lax.broadcasted_iota(jnp.int32, (n, NCONF), 1)
    swapped = jnp.where(jnp.bitwise_and(lane, 1) == 0,
                        jnp.roll(conf, -1, axis=1),
                        jnp.roll(conf, 1, axis=1))
    p8 = jax.nn.sigmoid(conf - swapped)
    pix = p_base + jax.lax.broadcasted_iota(jnp.int32, (n, 1), 0)
    xcol = jnp.bitwise_and(pix, fw - 1)
    yrow = jax.lax.shift_right_logical(pix, int(math.log2(fw)))
    scale = float(STEPS[i]) / float(IMG)
    cx = (xcol.astype(jnp.float32) + 0.5) * scale
    cy = (yrow.astype(jnp.float32) + 0.5) * scale
    pieces = []
    for k in range(NB):
        b = 14 * k
        pieces.append(p8[:, 2 * k:2 * k + 2])
        pieces.append(locv[:, 4 * k:4 * k + 4])
        pieces.append(cx)
        pieces.append(cy)
        pieces.append(jnp.broadcast_to(ctab[i:i + 1, b + 8:b + 14], (n, 6)))
    return jnp.concatenate(pieces, axis=1)


def _ssd_head_kernel(f0, f1, f2, f3, wa0, wb0, wc0, wa1, wb1, wc1, wa2, wb2,
                     wc2, wa3, wb3, wc3, btab, ctab, out_ref):
    j = pl.program_id(1)
    feats = (f0, f1, f2, f3)
    ws = ((wa0[...], wb0[...], wc0[...]), (wa1[...], wb1[...], wc1[...]),
          (wa2[...], wb2[...], wc2[...]), (wa3[...], wb3[...], wc3[...]))

    for c in range(4):
        @pl.when(j == c)
        def _l0(c=c):
            r0, r1 = c * 32, (c + 1) * 32
            acc = _conv_rows(feats[0], *ws[0], btab, 0, r0, r1)
            out_ref[0] = _assemble(acc, ctab, 0, c * PBLK)

    @pl.when(j == 4)
    def _l1():
        acc = _conv_rows(feats[1], *ws[1], btab, 1, 0, FHW[1])
        out_ref[0] = _assemble(acc, ctab, 1, 0)

    @pl.when(j == 5)
    def _l23():
        acc2 = _conv_rows(feats[2], *ws[2], btab, 2, 0, FHW[2])
        out_ref[0, 0:NPIX[2]] = _assemble(acc2, ctab, 2, 0)
        acc3 = _conv_rows(feats[3], *ws[3], btab, 3, 0, FHW[3])
        out_ref[0, NPIX[2]:NPIX[2] + NPIX[3]] = _assemble(acc3, ctab, 3, 0)


def kernel(feat0, feat1, feat2, feat3, Wc0, bc0, Wl0, bl0, Wc1, bc1, Wl1,
           bl1, Wc2, bc2, Wl2, bl2, Wc3, bc3, Wl3, bl3):
    B = feat0.shape[0]
    feats = (feat0, feat1, feat2, feat3)
    Wc = (Wc0, Wc1, Wc2, Wc3)
    bc = (bc0, bc1, bc2, bc3)
    Wl = (Wl0, Wl1, Wl2, Wl3)
    bl = (bl0, bl1, bl2, bl3)

    w_all = []
    for i in range(4):
        for kx in range(3):
            blocks = []
            for ky in range(3):
                blocks.append(Wc[i][ky, kx])
                blocks.append(Wl[i][ky, kx])
            w_all.append(jnp.concatenate(blocks, axis=-1).astype(jnp.bfloat16))

    btab = jnp.zeros((8, 128), jnp.float32)
    for i in range(4):
        btab = btab.at[i, 0:NCONF].set(bc[i])
        btab = btab.at[i, NCONF:NCH].set(bl[i])
    ctab = jnp.zeros((8, 128), jnp.float32)
    ctab = ctab.at[0:4, 0:NB * 14].set(jnp.asarray(_CONST56))

    last_use = (3, 4, 5, 5)

    def feat_spec(i):
        lu = last_use[i]
        return pl.BlockSpec(
            (1, FHW[i], FHW[i], CH[i]),
            lambda b, j, lu=lu: (jnp.minimum(b + (j > lu).astype(jnp.int32),
                                             B - 1), 0, 0, 0))

    def whole(arr):
        return pl.BlockSpec(arr.shape, lambda b, j: (0,) * arr.ndim)

    out = pl.pallas_call(
        _ssd_head_kernel,
        grid=(B, NBLK_ALL),
        in_specs=[feat_spec(i) for i in range(4)]
        + [whole(w) for w in w_all] + [whole(btab), whole(ctab)],
        out_specs=pl.BlockSpec((1, PBLK, NB * 14), lambda b, j: (b, j, 0)),
        out_shape=jax.ShapeDtypeStruct((B, NPIX_ALL, NB * 14), jnp.float32),
        compiler_params=pltpu.CompilerParams(
            dimension_semantics=("arbitrary", "arbitrary"),
            vmem_limit_bytes=128 * 1024 * 1024,
        ),
    )(*feats, *w_all, btab, ctab)
    return out.reshape(B, NPIX_ALL * NB, 14)


# speedup vs baseline: 4.6959x; 4.6959x over previous
import math

import jax
import jax.numpy as jnp
import numpy as np
from jax.experimental import pallas as pl
from jax.experimental.pallas import tpu as pltpu

IMG = 512
STEPS = (4, 8, 16, 32)
SCALES = (0.04, 0.1, 0.26, 0.45, 0.58)
FHW = (128, 64, 32, 16)
CH = (96, 192, 384, 768)
NPIX = tuple(f * f for f in FHW)
PBLK = 4096
NBLK_ALL = 6
NPIX_ALL = 21760
NB = 4
NCONF = NB * 2
NLOC = NB * 4
NCH = NCONF + NLOC


def _anchor_const_table() -> np.ndarray:
    tab = np.zeros((4, NB * 14), dtype=np.float32)
    for i in range(4):
        s, sn = SCALES[i], SCALES[i + 1]
        wh = [
            (s, s),
            (math.sqrt(s * sn), math.sqrt(s * sn)),
            (s * math.sqrt(2.0), s / math.sqrt(2.0)),
            (s * math.sqrt(0.5), s / math.sqrt(0.5)),
        ]
        for k in range(NB):
            base = 14 * k
            tab[i, base + 8] = wh[k][0]
            tab[i, base + 9] = wh[k][1]
            tab[i, base + 10:base + 14] = (0.1, 0.1, 0.2, 0.2)
    return tab


_CONST56 = _anchor_const_table()


def _conv_t(feat_ref, wt, btab_t, i, r0, r1):
    fw = FHW[i]
    fh = fw
    npx = (r1 - r0) * fw
    ulo, uhi = max(0, r0 - 1), min(fh, r1 + 1)
    nq = (uhi - ulo) * fw
    off = (r0 - ulo) * fw

    x = feat_ref[0, ulo:uhi].reshape(nq, CH[i]).astype(jnp.bfloat16)
    tt = jax.lax.dot_general(wt, x, (((1,), (1,)), ((), ())),
                             preferred_element_type=jnp.float32)

    lane = jax.lax.broadcasted_iota(jnp.int32, (1, npx), 1)
    xc = jnp.bitwise_and(lane, fw - 1)
    acc = jnp.broadcast_to(btab_t[0:NCH, i:i + 1], (NCH, npx))
    for ky in range(3):
        for kx in range(3):
            q = ky * 3 + kx
            s = (ky - 1) * fw + (kx - 1)
            st = off + s
            rows = tt[q * NCH:(q + 1) * NCH, :]
            if 0 <= st and st + npx <= nq:
                term = rows[:, st:st + npx] if st else rows[:, 0:npx]
            else:
                term = jnp.roll(rows, -st, axis=1)[:, 0:npx]
            mask = None
            if kx == 0:
                mask = xc > 0
            if kx == 2:
                mask = xc < fw - 1
            if ky == 0 and r0 == 0:
                m2 = lane >= fw
                mask = m2 if mask is None else (mask & m2)
            if ky == 2 and r1 == fh:
                m2 = lane < npx - fw
                mask = m2 if mask is None else (mask & m2)
            if mask is not None:
                term = jnp.where(mask, term, 0.0)
            acc = acc + term
    return acc


def _assemble_t(acc, ctab_t, i, p_base):
    npx = acc.shape[1]
    fw = FHW[i]
    conf = acc[0:NCONF, :]
    row = jax.lax.broadcasted_iota(jnp.int32, (NCONF, npx), 0)
    swapped = jnp.where(jnp.bitwise_and(row, 1) == 0,
                        jnp.roll(conf, -1, axis=0),
                        jnp.roll(conf, 1, axis=0))
    p8 = jax.nn.sigmoid(conf - swapped)
    lane = jax.lax.broadcasted_iota(jnp.int32, (1, npx), 1)
    xcol = jnp.bitwise_and(lane, fw - 1)
    yrow = jax.lax.shift_right_logical(p_base + lane, int(math.log2(fw)))
    scale = float(STEPS[i]) / float(IMG)
    cx = (xcol.astype(jnp.float32) + 0.5) * scale
    cy = (yrow.astype(jnp.float32) + 0.5) * scale
    pieces = []
    for k in range(NB):
        b = 14 * k
        pieces.append(p8[2 * k:2 * k + 2, :])
        pieces.append(acc[NCONF + 4 * k:NCONF + 4 * k + 4, :])
        pieces.append(cx)
        pieces.append(cy)
        pieces.append(jnp.broadcast_to(ctab_t[b + 8:b + 14, i:i + 1],
                                       (6, npx)))
    return jnp.concatenate(pieces, axis=0).T


def _ssd_head_kernel(f0, f1, f2, f3, wt0, wt1, wt2, wt3, btab_t, ctab_t,
                     out_ref):
    j = pl.program_id(1)
    feats = (f0, f1, f2, f3)
    wts = (wt0, wt1, wt2, wt3)
    bt = btab_t[...]
    ct = ctab_t[...]

    for c in range(4):
        @pl.when(j == c)
        def _l0(c=c):
            acc = _conv_t(feats[0], wts[0][...], bt, 0, c * 32, (c + 1) * 32)
            out_ref[0] = _assemble_t(acc, ct, 0, c * PBLK)

    @pl.when(j == 4)
    def _l1():
        acc = _conv_t(feats[1], wts[1][...], bt, 1, 0, FHW[1])
        out_ref[0] = _assemble_t(acc, ct, 1, 0)

    @pl.when(j == 5)
    def _l23():
        acc2 = _conv_t(feats[2], wts[2][...], bt, 2, 0, FHW[2])
        out_ref[0, 0:NPIX[2]] = _assemble_t(acc2, ct, 2, 0)
        acc3 = _conv_t(feats[3], wts[3][...], bt, 3, 0, FHW[3])
        out_ref[0, NPIX[2]:NPIX[2] + NPIX[3]] = _assemble_t(acc3, ct, 3, 0)


def kernel(feat0, feat1, feat2, feat3, Wc0, bc0, Wl0, bl0, Wc1, bc1, Wl1,
           bl1, Wc2, bc2, Wl2, bl2, Wc3, bc3, Wl3, bl3):
    B = feat0.shape[0]
    feats = (feat0, feat1, feat2, feat3)
    Wc = (Wc0, Wc1, Wc2, Wc3)
    bc = (bc0, bc1, bc2, bc3)
    Wl = (Wl0, Wl1, Wl2, Wl3)
    bl = (bl0, bl1, bl2, bl3)

    w_all = []
    for i in range(4):
        blocks = []
        for ky in range(3):
            for kx in range(3):
                blocks.append(Wc[i][ky, kx])
                blocks.append(Wl[i][ky, kx])
        w_all.append(
            jnp.concatenate(blocks, axis=-1).T.astype(jnp.bfloat16))

    btab_t = jnp.zeros((32, 8), jnp.float32)
    for i in range(4):
        btab_t = btab_t.at[0:NCONF, i].set(bc[i])
        btab_t = btab_t.at[NCONF:NCH, i].set(bl[i])
    ctab_t = jnp.zeros((64, 8), jnp.float32)
    ctab_t = ctab_t.at[0:NB * 14, 0:4].set(jnp.asarray(_CONST56).T)

    last_use = (3, 4, 5, 5)

    def feat_spec(i):
        lu = last_use[i]
        return pl.BlockSpec(
            (1, FHW[i], FHW[i], CH[i]),
            lambda b, j, lu=lu: (jnp.minimum(b + (j > lu).astype(jnp.int32),
                                             B - 1), 0, 0, 0))

    def whole(arr):
        return pl.BlockSpec(arr.shape, lambda b, j: (0,) * arr.ndim)

    out = pl.pallas_call(
        _ssd_head_kernel,
        grid=(B, NBLK_ALL),
        in_specs=[feat_spec(i) for i in range(4)]
        + [whole(w) for w in w_all] + [whole(btab_t), whole(ctab_t)],
        out_specs=pl.BlockSpec((1, PBLK, NB * 14), lambda b, j: (b, j, 0)),
        out_shape=jax.ShapeDtypeStruct((B, NPIX_ALL, NB * 14), jnp.float32),
        compiler_params=pltpu.CompilerParams(
            dimension_semantics=("arbitrary", "arbitrary"),
            vmem_limit_bytes=128 * 1024 * 1024,
        ),
    )(*feats, *w_all, btab_t, ctab_t)
    return out.reshape(B, NPIX_ALL * NB, 14)
